# Initial kernel scaffold; baseline (speedup 1.0000x reference)
import jax
import jax.numpy as jnp
from jax import lax
from jax.experimental import pallas as pl
from jax.experimental.pallas import tpu as pltpu
from jax.experimental.pallas import tpu_sc as plsc

N = 10000
CC = 16
GG = 64
E = 320000

N_PAD = 10240
BN = 1024
CH = 20736
SUB = CH // 2
E_PAD = 16 * CH
BB = 128
NBLK1 = CH // BB
NBLK2 = SUB // BB


def _logits(h, as_ref, ad_ref):
    h3 = h.reshape(BN // 128, 128, 128)
    a_s = as_ref[...].reshape(128)
    a_d = ad_ref[...].reshape(128)
    es = jnp.sum(h3 * a_s[None, None, :], axis=-1)
    ed = jnp.sum(h3 * a_d[None, None, :], axis=-1)
    return es, ed


def _dense_body_first(x_ref, w_ref, as_ref, ad_ref, h_ref, es_ref, ed_ref):
    h = jnp.dot(x_ref[...], w_ref[...], preferred_element_type=jnp.float32)
    h_ref[...] = h
    es_ref[...], ed_ref[...] = _logits(h, as_ref, ad_ref)


def _dense_body_mid(op_ref, b_ref, w_ref, as_ref, ad_ref, h_ref, es_ref,
                    ed_ref):
    act = jnp.maximum(op_ref[0] + op_ref[1] + b_ref[...].reshape(1, 128), 0.0)
    h = jnp.dot(act, w_ref[...], preferred_element_type=jnp.float32)
    h_ref[...] = h
    es_ref[...], ed_ref[...] = _logits(h, as_ref, ad_ref)


_DENSE_OUT_SPECS = [
    pl.BlockSpec((BN, 128), lambda i: (i, 0)),
    pl.BlockSpec((BN // 128, 128), lambda i: (i, 0)),
    pl.BlockSpec((BN // 128, 128), lambda i: (i, 0)),
]
_DENSE_OUT_SHAPE = [
    jax.ShapeDtypeStruct((N_PAD, 128), jnp.float32),
    jax.ShapeDtypeStruct((N_PAD // 128, 128), jnp.float32),
    jax.ShapeDtypeStruct((N_PAD // 128, 128), jnp.float32),
]


def _tc_dense_first(x_pad, w, a_s, a_d):
    return pl.pallas_call(
        _dense_body_first,
        grid=(N_PAD // BN,),
        in_specs=[
            pl.BlockSpec((BN, 128), lambda i: (i, 0)),
            pl.BlockSpec((128, 128), lambda i: (0, 0)),
            pl.BlockSpec((1, 128), lambda i: (0, 0)),
            pl.BlockSpec((1, 128), lambda i: (0, 0)),
        ],
        out_specs=_DENSE_OUT_SPECS,
        out_shape=_DENSE_OUT_SHAPE,
    )(x_pad, w, a_s.reshape(1, 128), a_d.reshape(1, 128))


def _tc_dense_mid(op, b, w, a_s, a_d):
    return pl.pallas_call(
        _dense_body_mid,
        grid=(N_PAD // BN,),
        in_specs=[
            pl.BlockSpec((2, BN, 128), lambda i: (0, i, 0)),
            pl.BlockSpec((1, 128), lambda i: (0, 0)),
            pl.BlockSpec((128, 128), lambda i: (0, 0)),
            pl.BlockSpec((1, 128), lambda i: (0, 0)),
            pl.BlockSpec((1, 128), lambda i: (0, 0)),
        ],
        out_specs=_DENSE_OUT_SPECS,
        out_shape=_DENSE_OUT_SHAPE,
    )(op, b.reshape(1, 128), w, a_s.reshape(1, 128), a_d.reshape(1, 128))


def _pool_body(op_ref, b_ref, batch_ref, hw1_ref, hb1_ref, hw2_ref, hb2_ref,
               out_ref, acc_ref, cnt_ref):
    i = pl.program_id(0)
    nsteps = pl.num_programs(0)

    @pl.when(i == 0)
    def _():
        acc_ref[...] = jnp.zeros_like(acc_ref)
        cnt_ref[...] = jnp.zeros_like(cnt_ref)

    h3 = op_ref[0] + op_ref[1] + b_ref[...].reshape(1, 1, 128)
    batch3 = batch_ref[...]
    iota3 = lax.broadcasted_iota(jnp.int32, (GG, BN // 128, 128), 0)
    oh2 = (iota3 == batch3[None, :, :]).astype(jnp.float32).reshape(GG, BN)
    h2 = h3.reshape(BN, 128)
    acc_ref[...] += jnp.dot(oh2, h2, preferred_element_type=jnp.float32)
    cnt_ref[...] += jnp.dot(oh2, jnp.ones((BN, 128), jnp.float32),
                            preferred_element_type=jnp.float32)

    @pl.when(i == nsteps - 1)
    def _():
        pooled = acc_ref[...] / jnp.maximum(cnt_ref[...], 1.0)
        z = jnp.dot(pooled, hw1_ref[...], preferred_element_type=jnp.float32)
        z = jnp.maximum(z + hb1_ref[...].reshape(1, 128), 0.0)
        out_ref[...] = (jnp.dot(z, hw2_ref[...],
                                preferred_element_type=jnp.float32)
                        + hb2_ref[...].reshape(1, CC))


def _tc_pool(op, b, batch2d, hw1, hb1, hw2, hb2):
    return pl.pallas_call(
        _pool_body,
        grid=(N_PAD // BN,),
        in_specs=[
            pl.BlockSpec((2, BN, 128), lambda i: (0, i, 0)),
            pl.BlockSpec((1, 128), lambda i: (0, 0)),
            pl.BlockSpec((BN // 128, 128), lambda i: (i, 0)),
            pl.BlockSpec((128, 128), lambda i: (0, 0)),
            pl.BlockSpec((1, 128), lambda i: (0, 0)),
            pl.BlockSpec((128, CC), lambda i: (0, 0)),
            pl.BlockSpec((1, CC), lambda i: (0, 0)),
        ],
        out_specs=pl.BlockSpec((GG, CC), lambda i: (0, 0)),
        out_shape=jax.ShapeDtypeStruct((GG, CC), jnp.float32),
        scratch_shapes=[
            pltpu.VMEM((GG, 128), jnp.float32),
            pltpu.VMEM((GG, 128), jnp.float32),
        ],
    )(op, b.reshape(1, 128), batch2d, hw1, hb1.reshape(1, 128), hw2,
      hb2.reshape(1, CC))


def _sc_edge_body(h_hbm, es_hbm, ed_hbm, src_hbm, dst_hbm, out_hbm,
                  es_v, ed_v, s_v, w_v, gidx, sidx, rows, alpha_v, sem,
                  s_acc, o_acc):
    c = lax.axis_index("c")
    t = lax.axis_index("s")

    pltpu.sync_copy(es_hbm, es_v)
    pltpu.sync_copy(ed_hbm, ed_v)

    def _mx(i, m):
        return jnp.maximum(m, es_v[pl.ds(i * 16, 16)])
    mvec = lax.fori_loop(0, N_PAD // 16, _mx,
                         jnp.full((16,), -3.0e38, jnp.float32))
    gvec = jnp.full((16,), jnp.max(mvec), jnp.float32)

    def _zrows(i, _):
        rows[0, pl.ds(i * 16, 16)] = jnp.zeros((16,), jnp.float32)
        return 0
    lax.fori_loop(0, 128 * 8, _zrows, 0)
    pltpu.sync_copy(rows.at[pl.ds(0, 5)], s_acc.at[pl.ds(t * 640, 640)])
    for k in range(5):
        pltpu.sync_copy(rows, o_acc.at[pl.ds(t * 640 + k * 128, 128)])
    plsc.subcore_barrier()

    base1 = t * CH

    def _p1(blk, _):
        eb = base1 + blk * BB
        pltpu.sync_copy(src_hbm.at[pl.ds(eb, BB)], gidx.at[0])
        pltpu.sync_copy(dst_hbm.at[pl.ds(eb, BB)], sidx.at[0])

        def _vreg(j, _2):
            sv = gidx[0, pl.ds(j * 16, 16)]
            dv = sidx[0, pl.ds(j * 16, 16)]
            esg = plsc.load_gather(es_v, [sv])
            edg = plsc.load_gather(ed_v, [dv])
            tsum = esg + edg
            e = jnp.where(tsum > 0, tsum, 0.2 * tsum)
            bn = gvec + edg
            bnd = jnp.where(bn > 0, bn, 0.2 * bn)
            w_v[pl.ds(blk * BB + j * 16, 16)] = jnp.exp(e - bnd)
            return 0
        lax.fori_loop(0, BB // 16, _vreg, 0)
        pltpu.sync_copy(w_v.at[pl.ds(blk * BB, BB)],
                        s_acc.at[sidx.at[0]], add=True)
        return 0
    lax.fori_loop(0, NBLK1, _p1, 0)

    plsc.subcore_barrier()
    pltpu.sync_copy(s_acc, s_v)

    base2 = base1 + c * SUB

    def _p2(blk, _):
        eb = base2 + blk * BB
        pltpu.sync_copy(src_hbm.at[pl.ds(eb, BB)], gidx.at[1])
        pltpu.sync_copy(dst_hbm.at[pl.ds(eb, BB)], sidx.at[1])

        def _al(j, _2):
            dv = sidx[1, pl.ds(j * 16, 16)]
            sg = plsc.load_gather(s_v, [dv])
            wv = w_v[pl.ds(c * SUB + blk * BB + j * 16, 16)]
            alpha_v[pl.ds(j * 16, 16)] = wv / (sg + 1e-16)
            return 0
        lax.fori_loop(0, BB // 16, _al, 0)

        pltpu.async_copy(h_hbm.at[gidx.at[1]], rows, sem).wait()

        def _scale(r, _2):
            av = plsc.load_gather(alpha_v, [jnp.full((16,), r, jnp.int32)])
            for kk in range(8):
                rows[r, pl.ds(kk * 16, 16)] = rows[r, pl.ds(kk * 16, 16)] * av
            return 0
        lax.fori_loop(0, BB, _scale, 0)

        pltpu.sync_copy(rows, o_acc.at[sidx.at[1]], add=True)
        return 0
    lax.fori_loop(0, NBLK2, _p2, 0)

    plsc.subcore_barrier()
    pltpu.sync_copy(o_acc.at[pl.ds(t * 640, 640)],
                    out_hbm.at[c, pl.ds(t * 640, 640)])


_sc_edge = pl.kernel(
    _sc_edge_body,
    out_type=jax.ShapeDtypeStruct((2, N_PAD, 128), jnp.float32),
    mesh=plsc.VectorSubcoreMesh(core_axis_name="c", subcore_axis_name="s"),
    scratch_types=[
        pltpu.VMEM((N_PAD,), jnp.float32),
        pltpu.VMEM((N_PAD,), jnp.float32),
        pltpu.VMEM((N_PAD,), jnp.float32),
        pltpu.VMEM((CH,), jnp.float32),
        pltpu.VMEM((2, BB), jnp.int32),
        pltpu.VMEM((2, BB), jnp.int32),
        pltpu.VMEM((BB, 128), jnp.float32),
        pltpu.VMEM((BB,), jnp.float32),
        pltpu.SemaphoreType.DMA,
        pltpu.VMEM_SHARED((N_PAD,), jnp.float32),
        pltpu.VMEM_SHARED((N_PAD, 128), jnp.float32),
    ],
)


def kernel(x, edge_index, batch_sample_indices,
           W0, asrc0, adst0, b0, W1, asrc1, adst1, b1, W2, asrc2, adst2, b2,
           hW1, hb1, hW2, hb2):
    n_extra = E_PAD - (E + N)
    pad_nodes = (jnp.arange(n_extra, dtype=jnp.int32) % (N_PAD - N - 1)) + N
    loop = jnp.arange(N, dtype=jnp.int32)
    src = jnp.concatenate([edge_index[0], loop, pad_nodes])
    dst = jnp.concatenate([edge_index[1], loop, pad_nodes])
    x_pad = jnp.pad(x, ((0, N_PAD - N), (0, 0)))
    batch2d = jnp.pad(batch_sample_indices.astype(jnp.int32),
                      (0, N_PAD - N), constant_values=GG).reshape(
                          N_PAD // 128, 128)

    h, es, ed = _tc_dense_first(x_pad, W0, asrc0, adst0)
    op = _sc_edge(h, es, ed, src, dst)
    h, es, ed = _tc_dense_mid(op, b0, W1, asrc1, adst1)
    op = _sc_edge(h, es, ed, src, dst)
    h, es, ed = _tc_dense_mid(op, b1, W2, asrc2, adst2)
    op = _sc_edge(h, es, ed, src, dst)
    return _tc_pool(op, b2, batch2d, hW1, hb1, hW2, hb2)


# speedup vs baseline: 18.3902x; 18.3902x over previous
import jax
import jax.numpy as jnp
from jax import lax
from jax.experimental import pallas as pl
from jax.experimental.pallas import tpu as pltpu
from jax.experimental.pallas import tpu_sc as plsc

N = 10000
CC = 16
GG = 64
E = 320000

N_PAD = 10240
BN = 1024
CH = 20736
SUB = CH // 2
E_PAD = 16 * CH
BB = 128
NBLK1 = CH // BB
NBLK2 = SUB // BB


def _logits(h, as_ref, ad_ref, es_ref, ed_ref, gm_ref, mx_ref):
    h3 = h.reshape(BN // 128, 128, 128)
    a_s = as_ref[...].reshape(128)
    a_d = ad_ref[...].reshape(128)
    es = jnp.sum(h3 * a_s[None, None, :], axis=-1)
    ed = jnp.sum(h3 * a_d[None, None, :], axis=-1)
    es_ref[...] = es
    ed_ref[...] = ed
    i = pl.program_id(0)
    m_blk = jnp.max(es)

    @pl.when(i == 0)
    def _():
        mx_ref[0, 0] = m_blk

    mx_ref[0, 0] = jnp.maximum(mx_ref[0, 0], m_blk)

    @pl.when(i == pl.num_programs(0) - 1)
    def _():
        gm_ref[...] = jnp.full((1, 128), mx_ref[0, 0], jnp.float32)


def _dense_body_first(x_ref, w_ref, as_ref, ad_ref, h_ref, es_ref, ed_ref,
                      gm_ref, mx_ref):
    h = jnp.dot(x_ref[...], w_ref[...], preferred_element_type=jnp.float32)
    h_ref[...] = h
    _logits(h, as_ref, ad_ref, es_ref, ed_ref, gm_ref, mx_ref)


def _dense_body_mid(op_ref, b_ref, w_ref, as_ref, ad_ref, h_ref, es_ref,
                    ed_ref, gm_ref, mx_ref):
    act = jnp.maximum(op_ref[0] + op_ref[1] + b_ref[...].reshape(1, 128), 0.0)
    h = jnp.dot(act, w_ref[...], preferred_element_type=jnp.float32)
    h_ref[...] = h
    _logits(h, as_ref, ad_ref, es_ref, ed_ref, gm_ref, mx_ref)


_DENSE_OUT_SPECS = [
    pl.BlockSpec((BN, 128), lambda i: (i, 0)),
    pl.BlockSpec((BN // 128, 128), lambda i: (i, 0)),
    pl.BlockSpec((BN // 128, 128), lambda i: (i, 0)),
    pl.BlockSpec((1, 128), lambda i: (0, 0)),
]
_DENSE_OUT_SHAPE = [
    jax.ShapeDtypeStruct((N_PAD, 128), jnp.float32),
    jax.ShapeDtypeStruct((N_PAD // 128, 128), jnp.float32),
    jax.ShapeDtypeStruct((N_PAD // 128, 128), jnp.float32),
    jax.ShapeDtypeStruct((1, 128), jnp.float32),
]
_DENSE_SCRATCH = [pltpu.SMEM((1, 1), jnp.float32)]


def _tc_dense_first(x_pad, w, a_s, a_d):
    return pl.pallas_call(
        _dense_body_first,
        grid=(N_PAD // BN,),
        in_specs=[
            pl.BlockSpec((BN, 128), lambda i: (i, 0)),
            pl.BlockSpec((128, 128), lambda i: (0, 0)),
            pl.BlockSpec((1, 128), lambda i: (0, 0)),
            pl.BlockSpec((1, 128), lambda i: (0, 0)),
        ],
        out_specs=_DENSE_OUT_SPECS,
        out_shape=_DENSE_OUT_SHAPE,
        scratch_shapes=_DENSE_SCRATCH,
    )(x_pad, w, a_s.reshape(1, 128), a_d.reshape(1, 128))


def _tc_dense_mid(op, b, w, a_s, a_d):
    return pl.pallas_call(
        _dense_body_mid,
        grid=(N_PAD // BN,),
        in_specs=[
            pl.BlockSpec((2, BN, 128), lambda i: (0, i, 0)),
            pl.BlockSpec((1, 128), lambda i: (0, 0)),
            pl.BlockSpec((128, 128), lambda i: (0, 0)),
            pl.BlockSpec((1, 128), lambda i: (0, 0)),
            pl.BlockSpec((1, 128), lambda i: (0, 0)),
        ],
        out_specs=_DENSE_OUT_SPECS,
        out_shape=_DENSE_OUT_SHAPE,
        scratch_shapes=_DENSE_SCRATCH,
    )(op, b.reshape(1, 128), w, a_s.reshape(1, 128), a_d.reshape(1, 128))


def _pool_body(op_ref, b_ref, batch_ref, hw1_ref, hb1_ref, hw2_ref, hb2_ref,
               out_ref, acc_ref, cnt_ref):
    i = pl.program_id(0)
    nsteps = pl.num_programs(0)

    @pl.when(i == 0)
    def _():
        acc_ref[...] = jnp.zeros_like(acc_ref)
        cnt_ref[...] = jnp.zeros_like(cnt_ref)

    h3 = op_ref[0] + op_ref[1] + b_ref[...].reshape(1, 1, 128)
    batch3 = batch_ref[...]
    iota3 = lax.broadcasted_iota(jnp.int32, (GG, BN // 128, 128), 0)
    oh2 = (iota3 == batch3[None, :, :]).astype(jnp.float32).reshape(GG, BN)
    h2 = h3.reshape(BN, 128)
    acc_ref[...] += jnp.dot(oh2, h2, preferred_element_type=jnp.float32)
    cnt_ref[...] += jnp.dot(oh2, jnp.ones((BN, 128), jnp.float32),
                            preferred_element_type=jnp.float32)

    @pl.when(i == nsteps - 1)
    def _():
        pooled = acc_ref[...] / jnp.maximum(cnt_ref[...], 1.0)
        z = jnp.dot(pooled, hw1_ref[...], preferred_element_type=jnp.float32)
        z = jnp.maximum(z + hb1_ref[...].reshape(1, 128), 0.0)
        out_ref[...] = (jnp.dot(z, hw2_ref[...],
                                preferred_element_type=jnp.float32)
                        + hb2_ref[...].reshape(1, CC))


def _tc_pool(op, b, batch2d, hw1, hb1, hw2, hb2):
    return pl.pallas_call(
        _pool_body,
        grid=(N_PAD // BN,),
        in_specs=[
            pl.BlockSpec((2, BN, 128), lambda i: (0, i, 0)),
            pl.BlockSpec((1, 128), lambda i: (0, 0)),
            pl.BlockSpec((BN // 128, 128), lambda i: (i, 0)),
            pl.BlockSpec((128, 128), lambda i: (0, 0)),
            pl.BlockSpec((1, 128), lambda i: (0, 0)),
            pl.BlockSpec((128, CC), lambda i: (0, 0)),
            pl.BlockSpec((1, CC), lambda i: (0, 0)),
        ],
        out_specs=pl.BlockSpec((GG, CC), lambda i: (0, 0)),
        out_shape=jax.ShapeDtypeStruct((GG, CC), jnp.float32),
        scratch_shapes=[
            pltpu.VMEM((GG, 128), jnp.float32),
            pltpu.VMEM((GG, 128), jnp.float32),
        ],
    )(op, b.reshape(1, 128), batch2d, hw1, hb1.reshape(1, 128), hw2,
      hb2.reshape(1, CC))


def _sc_edge_body(h_hbm, es_hbm, ed_hbm, gm_hbm, src_hbm, dst_hbm, out_hbm,
                  es_v, ed_v, s_v, gidx, sidx, rows, alpha_v, gm_v, sem,
                  s_acc, o_acc):
    c = lax.axis_index("c")
    t = lax.axis_index("s")

    def _unpack(hbm, flat):
        pltpu.sync_copy(hbm, rows.at[pl.ds(0, N_PAD // 128)])

        def _cp(i, _):
            for kk in range(8):
                flat[pl.ds(i * 128 + kk * 16, 16)] = rows[i, pl.ds(kk * 16, 16)]
            return 0
        lax.fori_loop(0, N_PAD // 128, _cp, 0)

    _unpack(es_hbm, es_v)
    _unpack(ed_hbm, ed_v)

    pltpu.sync_copy(gm_hbm, gm_v)
    gvec = gm_v[0, pl.ds(0, 16)]

    def _zrows(i, _):
        rows[0, pl.ds(i * 16, 16)] = jnp.zeros((16,), jnp.float32)
        return 0
    lax.fori_loop(0, 128 * 8, _zrows, 0)
    for k in range(5):
        pltpu.sync_copy(rows.at[0],
                        s_acc.at[pl.ds(t * 640 + k * 128, 128)])
        pltpu.sync_copy(rows, o_acc.at[pl.ds(t * 640 + k * 128, 128)])
    plsc.subcore_barrier()

    base1 = t * CH

    def _wvec(j, slot):
        sv = gidx[slot, pl.ds(j * 16, 16)]
        dv = sidx[slot, pl.ds(j * 16, 16)]
        esg = plsc.load_gather(es_v, [sv])
        edg = plsc.load_gather(ed_v, [dv])
        tsum = esg + edg
        e = jnp.where(tsum > 0, tsum, 0.2 * tsum)
        bn = gvec + edg
        bnd = jnp.where(bn > 0, bn, 0.2 * bn)
        return jnp.exp(e - bnd), dv

    def _p1(blk, _):
        eb = base1 + blk * BB
        pltpu.sync_copy(src_hbm.at[pl.ds(eb, BB)], gidx.at[0])
        pltpu.sync_copy(dst_hbm.at[pl.ds(eb, BB)], sidx.at[0])

        def _vreg(j, _2):
            wv, _dv = _wvec(j, 0)
            alpha_v[pl.ds(j * 16, 16)] = wv
            return 0
        lax.fori_loop(0, BB // 16, _vreg, 0)
        pltpu.sync_copy(alpha_v, s_acc.at[sidx.at[0]], add=True)
        return 0
    lax.fori_loop(0, NBLK1, _p1, 0)

    plsc.subcore_barrier()
    pltpu.sync_copy(s_acc, s_v)

    base2 = base1 + c * SUB

    def _p2(blk, _):
        eb = base2 + blk * BB
        pltpu.sync_copy(src_hbm.at[pl.ds(eb, BB)], gidx.at[1])
        pltpu.sync_copy(dst_hbm.at[pl.ds(eb, BB)], sidx.at[1])

        def _al(j, _2):
            wv, dv = _wvec(j, 1)
            sg = plsc.load_gather(s_v, [dv])
            alpha_v[pl.ds(j * 16, 16)] = wv / (sg + 1e-16)
            return 0
        lax.fori_loop(0, BB // 16, _al, 0)

        pltpu.async_copy(h_hbm.at[gidx.at[1]], rows, sem).wait()

        def _scale(r, _2):
            av = plsc.load_gather(alpha_v, [jnp.full((16,), r, jnp.int32)])
            for kk in range(8):
                rows[r, pl.ds(kk * 16, 16)] = rows[r, pl.ds(kk * 16, 16)] * av
            return 0
        lax.fori_loop(0, BB, _scale, 0)

        pltpu.sync_copy(rows, o_acc.at[sidx.at[1]], add=True)
        return 0
    lax.fori_loop(0, NBLK2, _p2, 0)

    plsc.subcore_barrier()
    pltpu.sync_copy(o_acc.at[pl.ds(t * 640, 640)],
                    out_hbm.at[c, pl.ds(t * 640, 640)])


_SC_EDGE_CACHE = []


def _sc_edge(h, es, ed, gm, src, dst):
    if not _SC_EDGE_CACHE:
        _SC_EDGE_CACHE.append(_make_sc_edge())
    return _SC_EDGE_CACHE[0](h, es, ed, gm, src, dst)


def _make_sc_edge():
    return pl.kernel(
        _sc_edge_body,
        out_type=jax.ShapeDtypeStruct((2, N_PAD, 128), jnp.float32),
        mesh=plsc.VectorSubcoreMesh(core_axis_name="c", subcore_axis_name="s",
                                    num_cores=2, num_subcores=16),
        compiler_params=pltpu.CompilerParams(needs_layout_passes=False),
        scratch_types=[
            pltpu.VMEM((N_PAD,), jnp.float32),
            pltpu.VMEM((N_PAD,), jnp.float32),
            pltpu.VMEM((N_PAD,), jnp.float32),
            pltpu.VMEM((2, BB), jnp.int32),
            pltpu.VMEM((2, BB), jnp.int32),
            pltpu.VMEM((BB, 128), jnp.float32),
            pltpu.VMEM((BB,), jnp.float32),
            pltpu.VMEM((1, 128), jnp.float32),
            pltpu.SemaphoreType.DMA,
            pltpu.VMEM_SHARED((N_PAD,), jnp.float32),
            pltpu.VMEM_SHARED((N_PAD, 128), jnp.float32),
        ],
    )


def kernel(x, edge_index, batch_sample_indices,
           W0, asrc0, adst0, b0, W1, asrc1, adst1, b1, W2, asrc2, adst2, b2,
           hW1, hb1, hW2, hb2):
    n_extra = E_PAD - (E + N)
    pad_nodes = (jnp.arange(n_extra, dtype=jnp.int32) % (N_PAD - N - 1)) + N
    loop = jnp.arange(N, dtype=jnp.int32)
    src = jnp.concatenate([edge_index[0], loop, pad_nodes])
    dst = jnp.concatenate([edge_index[1], loop, pad_nodes])
    x_pad = jnp.pad(x, ((0, N_PAD - N), (0, 0)))
    batch2d = jnp.pad(batch_sample_indices.astype(jnp.int32),
                      (0, N_PAD - N), constant_values=GG).reshape(
                          N_PAD // 128, 128)

    h, es, ed, gm = _tc_dense_first(x_pad, W0, asrc0, adst0)
    op = _sc_edge(h, es, ed, gm, src, dst)
    h, es, ed, gm = _tc_dense_mid(op, b0, W1, asrc1, adst1)
    op = _sc_edge(h, es, ed, gm, src, dst)
    h, es, ed, gm = _tc_dense_mid(op, b1, W2, asrc2, adst2)
    op = _sc_edge(h, es, ed, gm, src, dst)
    return _tc_pool(op, b2, batch2d, hW1, hb1, hW2, hb2)
